# baseline (speedup 1.0000x reference)
import jax
import jax.numpy as jnp
from jax.experimental import pallas as pl
from jax.experimental.pallas import tpu as pltpu

Q, K, D = 1024, 16384, 1536
BQ, BK = 256, 1024
NQ, NK = Q // BQ, K // BK
BK2 = 1024
NK2 = K // BK2
_INF = float("inf")
_EPS = 1e-12


def _stage1_body(q_ref, k_ref, mind_ref, sidx_ref, star_ref, sstar_ref,
                 fmin_ref, fidx_ref):
    i = pl.program_id(0)
    j = pl.program_id(1)
    q = q_ref[...]
    k = k_ref[...]
    qk = jax.lax.dot_general(q, k, (((1,), (1,)), ((), ())),
                             preferred_element_type=jnp.float32)
    k2 = jax.lax.dot_general(jnp.ones((1, D), jnp.float32), k * k,
                             (((1,), (1,)), ((), ())),
                             precision=jax.lax.Precision.HIGHEST,
                             preferred_element_type=jnp.float32)
    q2 = jnp.sum(q * q, axis=1, keepdims=True)
    d2 = (q2 - 2.0 * qk) + k2

    bmin = jnp.min(d2, axis=1, keepdims=True)
    io = jax.lax.broadcasted_iota(jnp.int32, (BQ, BK), 1) + j * BK
    barg = jnp.min(jnp.where(d2 == bmin, io, K), axis=1, keepdims=True)

    rows = pl.ds(i * BQ, BQ)

    @pl.when(j == 0)
    def _():
        fmin_ref[rows, :] = jnp.full((BQ, 1), _INF, jnp.float32)
        fidx_ref[rows, :] = jnp.zeros((BQ, 1), jnp.int32)

    old_min = fmin_ref[rows, :]
    old_idx = fidx_ref[rows, :]
    take = bmin < old_min
    new_min = jnp.where(take, bmin, old_min)
    new_idx = jnp.where(take, barg, old_idx)
    fmin_ref[rows, :] = new_min
    fidx_ref[rows, :] = new_idx

    @pl.when(j == NK - 1)
    def _():
        mind_ref[...] = jnp.sqrt(jnp.maximum(new_min, _EPS))

    @pl.when((i == NQ - 1) & (j == NK - 1))
    def _():
        fmin = fmin_ref[...]
        fidx = fidx_ref[...]
        s_val = jnp.max(fmin)
        qio = jax.lax.broadcasted_iota(jnp.int32, (Q, 1), 0)
        s_idx = jnp.min(jnp.where(fmin == s_val, qio, Q))
        star = jnp.sum(jnp.where(qio == s_idx, fidx, 0))
        sidx_ref[...] = jnp.full((1, 1), s_idx, jnp.int32)
        star_ref[...] = jnp.full((1, 1), star, jnp.int32)
        sstar_ref[...] = jnp.full((1, 1), jnp.sqrt(jnp.maximum(s_val, _EPS)),
                                  jnp.float32)


def _stage2_body(sidx_pref, star_pref, mt_ref, ms_ref, k_ref, sstar_ref,
                 out_ref, cand_ref):
    j = pl.program_id(0)
    k = k_ref[...]
    ms = ms_ref[...].reshape(1, D)
    mt = mt_ref[...].reshape(1, D)
    k2 = jax.lax.dot_general(jnp.ones((1, D), jnp.float32), k * k,
                             (((1,), (1,)), ((), ())),
                             precision=jax.lax.Precision.HIGHEST,
                             preferred_element_type=jnp.float32)
    msk = jax.lax.dot_general(ms, k, (((1,), (1,)), ((), ())),
                              preferred_element_type=jnp.float32)
    mtk = jax.lax.dot_general(mt, k, (((1,), (1,)), ((), ())),
                              precision=jax.lax.Precision.HIGHEST,
                              preferred_element_type=jnp.float32)
    m2s = jnp.sum(ms * ms)
    m2t = jnp.sum(mt * mt)
    ds = jnp.sqrt(jnp.maximum((k2 - 2.0 * msk) + m2s, _EPS))
    dt = jnp.sqrt(jnp.maximum((k2 - 2.0 * mtk) + m2t, _EPS))
    io = jax.lax.broadcasted_iota(jnp.int32, (1, BK2), 1)

    def top1(dvec):
        v = jnp.min(dvec)
        i1 = jnp.min(jnp.where(dvec == v, io, BK2))
        pay = jnp.sum(jnp.where(io == i1, dt, 0.0))
        return v, pay, i1

    bv1, bd1, i1 = top1(ds)
    ds_b = jnp.where(io == i1, _INF, ds)
    bv2, bd2, i2 = top1(ds_b)
    ds_c = jnp.where(io == i2, _INF, ds_b)
    bv3, bd3, _ = top1(ds_c)

    @pl.when(j == 0)
    def _():
        cand_ref[0] = _INF
        cand_ref[1] = _INF
        cand_ref[2] = _INF
        cand_ref[3] = 0.0
        cand_ref[4] = 0.0
        cand_ref[5] = 0.0

    rv1, rv2, rv3 = cand_ref[0], cand_ref[1], cand_ref[2]
    rd1, rd2, rd3 = cand_ref[3], cand_ref[4], cand_ref[5]

    c1 = bv1 < rv1
    o1v = jnp.where(c1, bv1, rv1)
    o1d = jnp.where(c1, bd1, rd1)
    a2 = bv1 < rv2
    A2v = jnp.where(a2, bv1, rv2)
    A2d = jnp.where(a2, bd1, rd2)
    A3v = jnp.where(a2, jnp.where(bv2 < rv2, bv2, rv2),
                    jnp.where(bv1 < rv3, bv1, rv3))
    A3d = jnp.where(a2, jnp.where(bv2 < rv2, bd2, rd2),
                    jnp.where(bv1 < rv3, bd1, rd3))
    b2c = bv2 < rv1
    B2v = jnp.where(b2c, bv2, rv1)
    B2d = jnp.where(b2c, bd2, rd1)
    B3v = jnp.where(b2c, jnp.where(bv3 < rv1, bv3, rv1),
                    jnp.where(bv2 < rv2, bv2, rv2))
    B3d = jnp.where(b2c, jnp.where(bv3 < rv1, bd3, rd1),
                    jnp.where(bv2 < rv2, bd2, rd2))
    o2v = jnp.where(c1, B2v, A2v)
    o2d = jnp.where(c1, B2d, A2d)
    o3v = jnp.where(c1, B3v, A3v)
    o3d = jnp.where(c1, B3d, A3d)
    cand_ref[0] = o1v
    cand_ref[1] = o2v
    cand_ref[2] = o3v
    cand_ref[3] = o1d
    cand_ref[4] = o2d
    cand_ref[5] = o3d

    @pl.when(j == NK2 - 1)
    def _():
        dc = jnp.sqrt(jnp.float32(D))
        s_star = sstar_ref[...]
        den = jnp.exp(jnp.full((1, 1), o2d) / dc) + \
            jnp.exp(jnp.full((1, 1), o3d) / dc)
        out_ref[...] = (1.0 - jnp.exp(s_star / dc) / den) * s_star


def kernel(queries, keys):
    min_d, s_idx, star_idx, s_star = pl.pallas_call(
        _stage1_body,
        grid=(NQ, NK),
        in_specs=[pl.BlockSpec((BQ, D), lambda i, j: (i, 0)),
                  pl.BlockSpec((BK, D), lambda i, j: (j, 0))],
        out_specs=[pl.BlockSpec((BQ, 1), lambda i, j: (i, 0)),
                   pl.BlockSpec((1, 1), lambda i, j: (0, 0)),
                   pl.BlockSpec((1, 1), lambda i, j: (0, 0)),
                   pl.BlockSpec((1, 1), lambda i, j: (0, 0))],
        out_shape=[jax.ShapeDtypeStruct((Q, 1), jnp.float32),
                   jax.ShapeDtypeStruct((1, 1), jnp.int32),
                   jax.ShapeDtypeStruct((1, 1), jnp.int32),
                   jax.ShapeDtypeStruct((1, 1), jnp.float32)],
        scratch_shapes=[pltpu.VMEM((Q, 1), jnp.float32),
                        pltpu.VMEM((Q, 1), jnp.int32)],
        compiler_params=pltpu.CompilerParams(
            dimension_semantics=("arbitrary", "arbitrary")),
    )(queries, keys)

    grid_spec = pltpu.PrefetchScalarGridSpec(
        num_scalar_prefetch=2,
        grid=(NK2,),
        in_specs=[pl.BlockSpec((1, 1, D), lambda j, s, t: (s[0], 0, 0)),
                  pl.BlockSpec((1, 1, D), lambda j, s, t: (t[0], 0, 0)),
                  pl.BlockSpec((BK2, D), lambda j, s, t: (j, 0)),
                  pl.BlockSpec((1, 1), lambda j, s, t: (0, 0))],
        out_specs=pl.BlockSpec((1, 1), lambda j, s, t: (0, 0)),
        scratch_shapes=[pltpu.SMEM((8,), jnp.float32)],
    )
    score = pl.pallas_call(
        _stage2_body,
        grid_spec=grid_spec,
        out_shape=jax.ShapeDtypeStruct((1, 1), jnp.float32),
        compiler_params=pltpu.CompilerParams(
            dimension_semantics=("arbitrary",)),
    )(s_idx.reshape((1,)), star_idx.reshape((1,)),
      queries.reshape(Q, 1, D), keys.reshape(K, 1, D), keys, s_star)

    return score[0, 0], min_d.reshape(32, 32)


# speedup vs baseline: 1.5698x; 1.5698x over previous
import jax
import jax.numpy as jnp
from jax.experimental import pallas as pl
from jax.experimental.pallas import tpu as pltpu

Q, K, D = 1024, 16384, 1536
BQ, BK = 256, 1024
NQ, NK = Q // BQ, K // BK
BK2 = 1024
NK2 = K // BK2
_INF = float("inf")
_EPS = 1e-12


def _stage1_body(q_ref, k_ref, mind_ref, sidx_ref, star_ref, sstar_ref,
                 fmin_ref, fidx_ref):
    j = pl.program_id(0)
    q = q_ref[...]
    k = k_ref[...]
    qk = jax.lax.dot_general(q, k, (((1,), (1,)), ((), ())),
                             preferred_element_type=jnp.float32)
    k2 = jax.lax.dot_general(jnp.ones((1, D), jnp.float32), k * k,
                             (((1,), (1,)), ((), ())),
                             precision=jax.lax.Precision.HIGHEST,
                             preferred_element_type=jnp.float32)
    e = k2 - 2.0 * qk
    bmin = jnp.min(e, axis=1, keepdims=True)
    io = jax.lax.broadcasted_iota(jnp.int32, (Q, BK), 1) + j * BK
    barg = jnp.min(jnp.where(e == bmin, io, K), axis=1, keepdims=True)

    @pl.when(j == 0)
    def _():
        fmin_ref[...] = jnp.full((Q, 1), _INF, jnp.float32)
        fidx_ref[...] = jnp.zeros((Q, 1), jnp.int32)

    old_min = fmin_ref[...]
    old_idx = fidx_ref[...]
    take = bmin < old_min
    new_min = jnp.where(take, bmin, old_min)
    new_idx = jnp.where(take, barg, old_idx)
    fmin_ref[...] = new_min
    fidx_ref[...] = new_idx

    @pl.when(j == NK - 1)
    def _():
        q2 = jnp.sum(q * q, axis=1, keepdims=True)
        d2min = q2 + new_min
        mind_ref[...] = jnp.sqrt(jnp.maximum(d2min, _EPS))
        s_val = jnp.max(d2min)
        qio = jax.lax.broadcasted_iota(jnp.int32, (Q, 1), 0)
        s_idx = jnp.min(jnp.where(d2min == s_val, qio, Q))
        star = jnp.sum(jnp.where(qio == s_idx, new_idx, 0))
        sidx_ref[...] = jnp.full((1, 1), s_idx, jnp.int32)
        star_ref[...] = jnp.full((1, 1), star, jnp.int32)
        sstar_ref[...] = jnp.full((1, 1), jnp.sqrt(jnp.maximum(s_val, _EPS)),
                                  jnp.float32)


def _stage2_body(sidx_pref, star_pref, mt_ref, ms_ref, k_ref, sstar_ref,
                 out_ref, cand_ref):
    j = pl.program_id(0)
    k = k_ref[...]
    ms = ms_ref[...].reshape(1, D)
    mt = mt_ref[...].reshape(1, D)
    k2 = jax.lax.dot_general(jnp.ones((1, D), jnp.float32), k * k,
                             (((1,), (1,)), ((), ())),
                             precision=jax.lax.Precision.HIGHEST,
                             preferred_element_type=jnp.float32)
    msk = jax.lax.dot_general(ms, k, (((1,), (1,)), ((), ())),
                              preferred_element_type=jnp.float32)
    mtk = jax.lax.dot_general(mt, k, (((1,), (1,)), ((), ())),
                              precision=jax.lax.Precision.HIGHEST,
                              preferred_element_type=jnp.float32)
    m2s = jnp.sum(ms * ms)
    m2t = jnp.sum(mt * mt)
    ds = jnp.sqrt(jnp.maximum((k2 - 2.0 * msk) + m2s, _EPS))
    dt = jnp.sqrt(jnp.maximum((k2 - 2.0 * mtk) + m2t, _EPS))
    io = jax.lax.broadcasted_iota(jnp.int32, (1, BK2), 1)

    def top1(dvec):
        v = jnp.min(dvec)
        i1 = jnp.min(jnp.where(dvec == v, io, BK2))
        pay = jnp.sum(jnp.where(io == i1, dt, 0.0))
        return v, pay, i1

    bv1, bd1, i1 = top1(ds)
    ds_b = jnp.where(io == i1, _INF, ds)
    bv2, bd2, i2 = top1(ds_b)
    ds_c = jnp.where(io == i2, _INF, ds_b)
    bv3, bd3, _ = top1(ds_c)

    @pl.when(j == 0)
    def _():
        cand_ref[0] = _INF
        cand_ref[1] = _INF
        cand_ref[2] = _INF
        cand_ref[3] = 0.0
        cand_ref[4] = 0.0
        cand_ref[5] = 0.0

    rv1, rv2, rv3 = cand_ref[0], cand_ref[1], cand_ref[2]
    rd1, rd2, rd3 = cand_ref[3], cand_ref[4], cand_ref[5]

    c1 = bv1 < rv1
    o1v = jnp.where(c1, bv1, rv1)
    o1d = jnp.where(c1, bd1, rd1)
    a2 = bv1 < rv2
    A2v = jnp.where(a2, bv1, rv2)
    A2d = jnp.where(a2, bd1, rd2)
    A3v = jnp.where(a2, jnp.where(bv2 < rv2, bv2, rv2),
                    jnp.where(bv1 < rv3, bv1, rv3))
    A3d = jnp.where(a2, jnp.where(bv2 < rv2, bd2, rd2),
                    jnp.where(bv1 < rv3, bd1, rd3))
    b2c = bv2 < rv1
    B2v = jnp.where(b2c, bv2, rv1)
    B2d = jnp.where(b2c, bd2, rd1)
    B3v = jnp.where(b2c, jnp.where(bv3 < rv1, bv3, rv1),
                    jnp.where(bv2 < rv2, bv2, rv2))
    B3d = jnp.where(b2c, jnp.where(bv3 < rv1, bd3, rd1),
                    jnp.where(bv2 < rv2, bd2, rd2))
    o2v = jnp.where(c1, B2v, A2v)
    o2d = jnp.where(c1, B2d, A2d)
    o3v = jnp.where(c1, B3v, A3v)
    o3d = jnp.where(c1, B3d, A3d)
    cand_ref[0] = o1v
    cand_ref[1] = o2v
    cand_ref[2] = o3v
    cand_ref[3] = o1d
    cand_ref[4] = o2d
    cand_ref[5] = o3d

    @pl.when(j == NK2 - 1)
    def _():
        dc = jnp.sqrt(jnp.float32(D))
        s_star = sstar_ref[...]
        den = jnp.exp(jnp.full((1, 1), o2d) / dc) + \
            jnp.exp(jnp.full((1, 1), o3d) / dc)
        out_ref[...] = (1.0 - jnp.exp(s_star / dc) / den) * s_star


def kernel(queries, keys):
    min_d, s_idx, star_idx, s_star = pl.pallas_call(
        _stage1_body,
        grid=(NK,),
        in_specs=[pl.BlockSpec((Q, D), lambda j: (0, 0)),
                  pl.BlockSpec((BK, D), lambda j: (j, 0))],
        out_specs=[pl.BlockSpec((Q, 1), lambda j: (0, 0)),
                   pl.BlockSpec((1, 1), lambda j: (0, 0)),
                   pl.BlockSpec((1, 1), lambda j: (0, 0)),
                   pl.BlockSpec((1, 1), lambda j: (0, 0))],
        out_shape=[jax.ShapeDtypeStruct((Q, 1), jnp.float32),
                   jax.ShapeDtypeStruct((1, 1), jnp.int32),
                   jax.ShapeDtypeStruct((1, 1), jnp.int32),
                   jax.ShapeDtypeStruct((1, 1), jnp.float32)],
        scratch_shapes=[pltpu.VMEM((Q, 1), jnp.float32),
                        pltpu.VMEM((Q, 1), jnp.int32)],
        compiler_params=pltpu.CompilerParams(
            dimension_semantics=("arbitrary",)),
    )(queries, keys)

    grid_spec = pltpu.PrefetchScalarGridSpec(
        num_scalar_prefetch=2,
        grid=(NK2,),
        in_specs=[pl.BlockSpec((1, 1, D), lambda j, s, t: (s[0], 0, 0)),
                  pl.BlockSpec((1, 1, D), lambda j, s, t: (t[0], 0, 0)),
                  pl.BlockSpec((BK2, D), lambda j, s, t: (j, 0)),
                  pl.BlockSpec((1, 1), lambda j, s, t: (0, 0))],
        out_specs=pl.BlockSpec((1, 1), lambda j, s, t: (0, 0)),
        scratch_shapes=[pltpu.SMEM((8,), jnp.float32)],
    )
    score = pl.pallas_call(
        _stage2_body,
        grid_spec=grid_spec,
        out_shape=jax.ShapeDtypeStruct((1, 1), jnp.float32),
        compiler_params=pltpu.CompilerParams(
            dimension_semantics=("arbitrary",)),
    )(s_idx.reshape((1,)), star_idx.reshape((1,)),
      queries.reshape(Q, 1, D), keys.reshape(K, 1, D), keys, s_star)

    return score[0, 0], min_d.reshape(32, 32)


# speedup vs baseline: 4.4587x; 2.8403x over previous
import jax
import jax.numpy as jnp
from jax.experimental import pallas as pl
from jax.experimental.pallas import tpu as pltpu

Q, K, D = 1024, 16384, 1536
BQ, BK = 256, 1024
NQ, NK = Q // BQ, K // BK
BK2 = 1024
NK2 = K // BK2
_INF = float("inf")
_EPS = 1e-12


def _stage1_body(q_ref, k_ref, mind_ref, sidx_ref, star_ref, sstar_ref,
                 fmin_ref, fidx_ref):
    j = pl.program_id(0)
    q = q_ref[...]
    k = k_ref[...]
    qk = jax.lax.dot_general(q, k, (((1,), (1,)), ((), ())),
                             preferred_element_type=jnp.float32)
    k2 = jax.lax.dot_general(jnp.ones((1, D), jnp.float32), k * k,
                             (((1,), (1,)), ((), ())),
                             preferred_element_type=jnp.float32)
    e = k2 - 2.0 * qk
    bmin = jnp.min(e, axis=1, keepdims=True)
    io = jax.lax.broadcasted_iota(jnp.int32, (Q, BK), 1) + j * BK
    barg = jnp.min(jnp.where(e == bmin, io, K), axis=1, keepdims=True)

    @pl.when(j == 0)
    def _():
        fmin_ref[...] = jnp.full((Q, 1), _INF, jnp.float32)
        fidx_ref[...] = jnp.zeros((Q, 1), jnp.int32)

    old_min = fmin_ref[...]
    old_idx = fidx_ref[...]
    take = bmin < old_min
    new_min = jnp.where(take, bmin, old_min)
    new_idx = jnp.where(take, barg, old_idx)
    fmin_ref[...] = new_min
    fidx_ref[...] = new_idx

    @pl.when(j == NK - 1)
    def _():
        q2 = jnp.sum(q * q, axis=1, keepdims=True)
        d2min = q2 + new_min
        mind_ref[...] = jnp.sqrt(jnp.maximum(d2min, _EPS))
        s_val = jnp.max(d2min)
        qio = jax.lax.broadcasted_iota(jnp.int32, (Q, 1), 0)
        s_idx = jnp.min(jnp.where(d2min == s_val, qio, Q))
        star = jnp.sum(jnp.where(qio == s_idx, new_idx, 0))
        sidx_ref[...] = jnp.full((1, 1), s_idx, jnp.int32)
        star_ref[...] = jnp.full((1, 1), star, jnp.int32)
        sstar_ref[...] = jnp.full((1, 1), jnp.sqrt(jnp.maximum(s_val, _EPS)),
                                  jnp.float32)


def _stage2_body(mt_ref, ms_ref, k_ref, sstar_ref, out_ref, cand_ref):
    j = pl.program_id(0)
    k = k_ref[...]
    ms = ms_ref[...]
    mt = mt_ref[...]
    dks = k - ms
    dkt = k - mt
    ds = jnp.sqrt(jnp.maximum(
        jnp.sum(dks * dks, axis=1, keepdims=True), _EPS))
    dt = jnp.sqrt(jnp.maximum(
        jnp.sum(dkt * dkt, axis=1, keepdims=True), _EPS))
    io = jax.lax.broadcasted_iota(jnp.int32, (BK2, 1), 0)

    def top1(dvec):
        v = jnp.min(dvec)
        i1 = jnp.min(jnp.where(dvec == v, io, BK2))
        pay = jnp.sum(jnp.where(io == i1, dt, 0.0))
        return v, pay, i1

    bv1, bd1, i1 = top1(ds)
    ds_b = jnp.where(io == i1, _INF, ds)
    bv2, bd2, i2 = top1(ds_b)
    ds_c = jnp.where(io == i2, _INF, ds_b)
    bv3, bd3, _ = top1(ds_c)

    @pl.when(j == 0)
    def _():
        cand_ref[0] = _INF
        cand_ref[1] = _INF
        cand_ref[2] = _INF
        cand_ref[3] = 0.0
        cand_ref[4] = 0.0
        cand_ref[5] = 0.0

    rv1, rv2, rv3 = cand_ref[0], cand_ref[1], cand_ref[2]
    rd1, rd2, rd3 = cand_ref[3], cand_ref[4], cand_ref[5]

    c1 = bv1 < rv1
    o1v = jnp.where(c1, bv1, rv1)
    o1d = jnp.where(c1, bd1, rd1)
    a2 = bv1 < rv2
    A2v = jnp.where(a2, bv1, rv2)
    A2d = jnp.where(a2, bd1, rd2)
    A3v = jnp.where(a2, jnp.where(bv2 < rv2, bv2, rv2),
                    jnp.where(bv1 < rv3, bv1, rv3))
    A3d = jnp.where(a2, jnp.where(bv2 < rv2, bd2, rd2),
                    jnp.where(bv1 < rv3, bd1, rd3))
    b2c = bv2 < rv1
    B2v = jnp.where(b2c, bv2, rv1)
    B2d = jnp.where(b2c, bd2, rd1)
    B3v = jnp.where(b2c, jnp.where(bv3 < rv1, bv3, rv1),
                    jnp.where(bv2 < rv2, bv2, rv2))
    B3d = jnp.where(b2c, jnp.where(bv3 < rv1, bd3, rd1),
                    jnp.where(bv2 < rv2, bd2, rd2))
    o2v = jnp.where(c1, B2v, A2v)
    o2d = jnp.where(c1, B2d, A2d)
    o3v = jnp.where(c1, B3v, A3v)
    o3d = jnp.where(c1, B3d, A3d)
    cand_ref[0] = o1v
    cand_ref[1] = o2v
    cand_ref[2] = o3v
    cand_ref[3] = o1d
    cand_ref[4] = o2d
    cand_ref[5] = o3d

    @pl.when(j == NK2 - 1)
    def _():
        dc = jnp.sqrt(jnp.float32(D))
        s_star = sstar_ref[...]
        den = jnp.exp(jnp.full((1, 1), o2d) / dc) + \
            jnp.exp(jnp.full((1, 1), o3d) / dc)
        out_ref[...] = (1.0 - jnp.exp(s_star / dc) / den) * s_star


def kernel(queries, keys):
    min_d, s_idx, star_idx, s_star = pl.pallas_call(
        _stage1_body,
        grid=(NK,),
        in_specs=[pl.BlockSpec((Q, D), lambda j: (0, 0)),
                  pl.BlockSpec((BK, D), lambda j: (j, 0))],
        out_specs=[pl.BlockSpec((Q, 1), lambda j: (0, 0)),
                   pl.BlockSpec((1, 1), lambda j: (0, 0)),
                   pl.BlockSpec((1, 1), lambda j: (0, 0)),
                   pl.BlockSpec((1, 1), lambda j: (0, 0))],
        out_shape=[jax.ShapeDtypeStruct((Q, 1), jnp.float32),
                   jax.ShapeDtypeStruct((1, 1), jnp.int32),
                   jax.ShapeDtypeStruct((1, 1), jnp.int32),
                   jax.ShapeDtypeStruct((1, 1), jnp.float32)],
        scratch_shapes=[pltpu.VMEM((Q, 1), jnp.float32),
                        pltpu.VMEM((Q, 1), jnp.int32)],
        compiler_params=pltpu.CompilerParams(
            dimension_semantics=("arbitrary",)),
    )(queries, keys)

    m_test = jax.lax.dynamic_slice(queries, (s_idx[0, 0], 0), (1, D))
    m_star = jax.lax.dynamic_slice(keys, (star_idx[0, 0], 0), (1, D))
    score = pl.pallas_call(
        _stage2_body,
        grid=(NK2,),
        in_specs=[pl.BlockSpec((1, D), lambda j: (0, 0)),
                  pl.BlockSpec((1, D), lambda j: (0, 0)),
                  pl.BlockSpec((BK2, D), lambda j: (j, 0)),
                  pl.BlockSpec((1, 1), lambda j: (0, 0))],
        out_specs=pl.BlockSpec((1, 1), lambda j: (0, 0)),
        out_shape=jax.ShapeDtypeStruct((1, 1), jnp.float32),
        scratch_shapes=[pltpu.SMEM((8,), jnp.float32)],
        compiler_params=pltpu.CompilerParams(
            dimension_semantics=("arbitrary",)),
    )(m_test, m_star, keys, s_star)

    return score[0, 0], min_d.reshape(32, 32)


# speedup vs baseline: 4.9650x; 1.1135x over previous
import jax
import jax.numpy as jnp
from jax.experimental import pallas as pl
from jax.experimental.pallas import tpu as pltpu

Q, K, D = 1024, 16384, 1536
BQ, BK = 256, 1024
NQ, NK = Q // BQ, K // BK
BK2 = 1024
NK2 = K // BK2
_INF = float("inf")
_EPS = 1e-12


def _stage1_body(q_ref, k_ref, mind_ref, sidx_ref, star_ref, sstar_ref,
                 fmin_ref, fidx_ref):
    j = pl.program_id(0)
    q = q_ref[...]
    k = k_ref[...]
    kq = jax.lax.dot_general(k, q, (((1,), (1,)), ((), ())),
                             preferred_element_type=jnp.float32)
    k2 = jnp.sum(k * k, axis=1, keepdims=True)
    e = k2 - 2.0 * kq
    bmin = jnp.min(e, axis=0, keepdims=True)
    io = jax.lax.broadcasted_iota(jnp.int32, (BK, Q), 0) + j * BK
    barg = jnp.min(jnp.where(e == bmin, io, K), axis=0, keepdims=True)

    @pl.when(j == 0)
    def _():
        fmin_ref[...] = jnp.full((1, Q), _INF, jnp.float32)
        fidx_ref[...] = jnp.zeros((1, Q), jnp.int32)

    old_min = fmin_ref[...]
    old_idx = fidx_ref[...]
    take = bmin < old_min
    new_min = jnp.where(take, bmin, old_min)
    new_idx = jnp.where(take, barg, old_idx)
    fmin_ref[...] = new_min
    fidx_ref[...] = new_idx

    @pl.when(j == NK - 1)
    def _():
        q2 = jax.lax.dot_general(jnp.ones((1, D), jnp.float32), q * q,
                                 (((1,), (1,)), ((), ())),
                                 precision=jax.lax.Precision.HIGHEST,
                                 preferred_element_type=jnp.float32)
        d2min = q2 + new_min
        mind_ref[...] = jnp.sqrt(jnp.maximum(d2min, _EPS))
        s_val = jnp.max(d2min)
        qio = jax.lax.broadcasted_iota(jnp.int32, (1, Q), 1)
        s_idx = jnp.min(jnp.where(d2min == s_val, qio, Q))
        star = jnp.sum(jnp.where(qio == s_idx, new_idx, 0))
        sidx_ref[...] = jnp.full((1, 1), s_idx, jnp.int32)
        star_ref[...] = jnp.full((1, 1), star, jnp.int32)
        sstar_ref[...] = jnp.full((1, 1), jnp.sqrt(jnp.maximum(s_val, _EPS)),
                                  jnp.float32)


def _stage2_body(mt_ref, ms_ref, k_ref, sstar_ref, out_ref, cand_ref):
    j = pl.program_id(0)
    k = k_ref[...]
    ms = ms_ref[...]
    mt = mt_ref[...]
    dks = k - ms
    dkt = k - mt
    ds = jnp.sqrt(jnp.maximum(
        jnp.sum(dks * dks, axis=1, keepdims=True), _EPS))
    dt = jnp.sqrt(jnp.maximum(
        jnp.sum(dkt * dkt, axis=1, keepdims=True), _EPS))
    io = jax.lax.broadcasted_iota(jnp.int32, (BK2, 1), 0)

    def top1(dvec):
        v = jnp.min(dvec)
        i1 = jnp.min(jnp.where(dvec == v, io, BK2))
        pay = jnp.sum(jnp.where(io == i1, dt, 0.0))
        return v, pay, i1

    bv1, bd1, i1 = top1(ds)
    ds_b = jnp.where(io == i1, _INF, ds)
    bv2, bd2, i2 = top1(ds_b)
    ds_c = jnp.where(io == i2, _INF, ds_b)
    bv3, bd3, _ = top1(ds_c)

    @pl.when(j == 0)
    def _():
        cand_ref[0] = _INF
        cand_ref[1] = _INF
        cand_ref[2] = _INF
        cand_ref[3] = 0.0
        cand_ref[4] = 0.0
        cand_ref[5] = 0.0

    rv1, rv2, rv3 = cand_ref[0], cand_ref[1], cand_ref[2]
    rd1, rd2, rd3 = cand_ref[3], cand_ref[4], cand_ref[5]

    c1 = bv1 < rv1
    o1v = jnp.where(c1, bv1, rv1)
    o1d = jnp.where(c1, bd1, rd1)
    a2 = bv1 < rv2
    A2v = jnp.where(a2, bv1, rv2)
    A2d = jnp.where(a2, bd1, rd2)
    A3v = jnp.where(a2, jnp.where(bv2 < rv2, bv2, rv2),
                    jnp.where(bv1 < rv3, bv1, rv3))
    A3d = jnp.where(a2, jnp.where(bv2 < rv2, bd2, rd2),
                    jnp.where(bv1 < rv3, bd1, rd3))
    b2c = bv2 < rv1
    B2v = jnp.where(b2c, bv2, rv1)
    B2d = jnp.where(b2c, bd2, rd1)
    B3v = jnp.where(b2c, jnp.where(bv3 < rv1, bv3, rv1),
                    jnp.where(bv2 < rv2, bv2, rv2))
    B3d = jnp.where(b2c, jnp.where(bv3 < rv1, bd3, rd1),
                    jnp.where(bv2 < rv2, bd2, rd2))
    o2v = jnp.where(c1, B2v, A2v)
    o2d = jnp.where(c1, B2d, A2d)
    o3v = jnp.where(c1, B3v, A3v)
    o3d = jnp.where(c1, B3d, A3d)
    cand_ref[0] = o1v
    cand_ref[1] = o2v
    cand_ref[2] = o3v
    cand_ref[3] = o1d
    cand_ref[4] = o2d
    cand_ref[5] = o3d

    @pl.when(j == NK2 - 1)
    def _():
        dc = jnp.sqrt(jnp.float32(D))
        s_star = sstar_ref[...]
        den = jnp.exp(jnp.full((1, 1), o2d) / dc) + \
            jnp.exp(jnp.full((1, 1), o3d) / dc)
        out_ref[...] = (1.0 - jnp.exp(s_star / dc) / den) * s_star


def kernel(queries, keys):
    min_d, s_idx, star_idx, s_star = pl.pallas_call(
        _stage1_body,
        grid=(NK,),
        in_specs=[pl.BlockSpec((Q, D), lambda j: (0, 0)),
                  pl.BlockSpec((BK, D), lambda j: (j, 0))],
        out_specs=[pl.BlockSpec((1, Q), lambda j: (0, 0)),
                   pl.BlockSpec((1, 1), lambda j: (0, 0)),
                   pl.BlockSpec((1, 1), lambda j: (0, 0)),
                   pl.BlockSpec((1, 1), lambda j: (0, 0))],
        out_shape=[jax.ShapeDtypeStruct((1, Q), jnp.float32),
                   jax.ShapeDtypeStruct((1, 1), jnp.int32),
                   jax.ShapeDtypeStruct((1, 1), jnp.int32),
                   jax.ShapeDtypeStruct((1, 1), jnp.float32)],
        scratch_shapes=[pltpu.VMEM((1, Q), jnp.float32),
                        pltpu.VMEM((1, Q), jnp.int32)],
        compiler_params=pltpu.CompilerParams(
            dimension_semantics=("arbitrary",)),
    )(queries, keys)

    m_test = jax.lax.dynamic_slice(queries, (s_idx[0, 0], 0), (1, D))
    m_star = jax.lax.dynamic_slice(keys, (star_idx[0, 0], 0), (1, D))
    score = pl.pallas_call(
        _stage2_body,
        grid=(NK2,),
        in_specs=[pl.BlockSpec((1, D), lambda j: (0, 0)),
                  pl.BlockSpec((1, D), lambda j: (0, 0)),
                  pl.BlockSpec((BK2, D), lambda j: (j, 0)),
                  pl.BlockSpec((1, 1), lambda j: (0, 0))],
        out_specs=pl.BlockSpec((1, 1), lambda j: (0, 0)),
        out_shape=jax.ShapeDtypeStruct((1, 1), jnp.float32),
        scratch_shapes=[pltpu.SMEM((8,), jnp.float32)],
        compiler_params=pltpu.CompilerParams(
            dimension_semantics=("arbitrary",)),
    )(m_test, m_star, keys, s_star)

    return score[0, 0], min_d.reshape(32, 32)


# speedup vs baseline: 5.2114x; 1.0496x over previous
import jax
import jax.numpy as jnp
from jax.experimental import pallas as pl
from jax.experimental.pallas import tpu as pltpu

Q, K, D = 1024, 16384, 1536
BQ, BK = 256, 1024
NQ, NK = Q // BQ, K // BK
BK2 = 1024
NK2 = K // BK2
_INF = float("inf")
_EPS = 1e-12


def _stage1_body(q_ref, k_ref, mind_ref, sidx_ref, bstep_ref, sstar_ref,
                 fmin_ref, fstep_ref):
    j = pl.program_id(0)
    q = q_ref[...]
    k = k_ref[...]
    kq = jax.lax.dot_general(k, q, (((1,), (1,)), ((), ())),
                             preferred_element_type=jnp.float32)
    k2 = jnp.sum(k * k, axis=1, keepdims=True)
    bmin = jnp.min(k2 - 2.0 * kq, axis=0, keepdims=True)

    @pl.when(j == 0)
    def _():
        fmin_ref[...] = jnp.full((1, Q), _INF, jnp.float32)
        fstep_ref[...] = jnp.zeros((1, Q), jnp.int32)

    old_min = fmin_ref[...]
    old_step = fstep_ref[...]
    take = bmin < old_min
    new_min = jnp.where(take, bmin, old_min)
    new_step = jnp.where(take, j, old_step)
    fmin_ref[...] = new_min
    fstep_ref[...] = new_step

    @pl.when(j == NK - 1)
    def _():
        q2 = jax.lax.dot_general(jnp.ones((1, D), jnp.float32), q * q,
                                 (((1,), (1,)), ((), ())),
                                 precision=jax.lax.Precision.HIGHEST,
                                 preferred_element_type=jnp.float32)
        d2min = q2 + new_min
        mind_ref[...] = jnp.sqrt(jnp.maximum(d2min, _EPS))
        s_val = jnp.max(d2min)
        qio = jax.lax.broadcasted_iota(jnp.int32, (1, Q), 1)
        s_idx = jnp.min(jnp.where(d2min == s_val, qio, Q))
        bstep = jnp.sum(jnp.where(qio == s_idx, new_step, 0))
        sidx_ref[...] = jnp.full((1, 1), s_idx, jnp.int32)
        bstep_ref[...] = jnp.full((1, 1), bstep, jnp.int32)
        sstar_ref[...] = jnp.full((1, 1), jnp.sqrt(jnp.maximum(s_val, _EPS)),
                                  jnp.float32)


def _star_body(bstep_pref, mt_ref, k_ref, star_ref):
    k = k_ref[...]
    mt = mt_ref[...]
    kq = jax.lax.dot_general(k, mt, (((1,), (1,)), ((), ())),
                             preferred_element_type=jnp.float32)
    k2 = jnp.sum(k * k, axis=1, keepdims=True)
    e = k2 - 2.0 * kq
    v = jnp.min(e)
    io = jax.lax.broadcasted_iota(jnp.int32, (BK, 1), 0)
    loc = jnp.min(jnp.where(e == v, io, BK))
    star_ref[...] = jnp.full((1, 1), loc + bstep_pref[0] * BK, jnp.int32)


def _stage2_body(mt_ref, ms_ref, k_ref, sstar_ref, out_ref, cand_ref):
    j = pl.program_id(0)
    k = k_ref[...]
    ms = ms_ref[...]
    mt = mt_ref[...]
    dks = k - ms
    dkt = k - mt
    ds = jnp.sqrt(jnp.maximum(
        jnp.sum(dks * dks, axis=1, keepdims=True), _EPS))
    dt = jnp.sqrt(jnp.maximum(
        jnp.sum(dkt * dkt, axis=1, keepdims=True), _EPS))
    io = jax.lax.broadcasted_iota(jnp.int32, (BK2, 1), 0)

    def top1(dvec):
        v = jnp.min(dvec)
        i1 = jnp.min(jnp.where(dvec == v, io, BK2))
        pay = jnp.sum(jnp.where(io == i1, dt, 0.0))
        return v, pay, i1

    bv1, bd1, i1 = top1(ds)
    ds_b = jnp.where(io == i1, _INF, ds)
    bv2, bd2, i2 = top1(ds_b)
    ds_c = jnp.where(io == i2, _INF, ds_b)
    bv3, bd3, _ = top1(ds_c)

    @pl.when(j == 0)
    def _():
        cand_ref[0] = _INF
        cand_ref[1] = _INF
        cand_ref[2] = _INF
        cand_ref[3] = 0.0
        cand_ref[4] = 0.0
        cand_ref[5] = 0.0

    rv1, rv2, rv3 = cand_ref[0], cand_ref[1], cand_ref[2]
    rd1, rd2, rd3 = cand_ref[3], cand_ref[4], cand_ref[5]

    c1 = bv1 < rv1
    o1v = jnp.where(c1, bv1, rv1)
    o1d = jnp.where(c1, bd1, rd1)
    a2 = bv1 < rv2
    A2v = jnp.where(a2, bv1, rv2)
    A2d = jnp.where(a2, bd1, rd2)
    A3v = jnp.where(a2, jnp.where(bv2 < rv2, bv2, rv2),
                    jnp.where(bv1 < rv3, bv1, rv3))
    A3d = jnp.where(a2, jnp.where(bv2 < rv2, bd2, rd2),
                    jnp.where(bv1 < rv3, bd1, rd3))
    b2c = bv2 < rv1
    B2v = jnp.where(b2c, bv2, rv1)
    B2d = jnp.where(b2c, bd2, rd1)
    B3v = jnp.where(b2c, jnp.where(bv3 < rv1, bv3, rv1),
                    jnp.where(bv2 < rv2, bv2, rv2))
    B3d = jnp.where(b2c, jnp.where(bv3 < rv1, bd3, rd1),
                    jnp.where(bv2 < rv2, bd2, rd2))
    o2v = jnp.where(c1, B2v, A2v)
    o2d = jnp.where(c1, B2d, A2d)
    o3v = jnp.where(c1, B3v, A3v)
    o3d = jnp.where(c1, B3d, A3d)
    cand_ref[0] = o1v
    cand_ref[1] = o2v
    cand_ref[2] = o3v
    cand_ref[3] = o1d
    cand_ref[4] = o2d
    cand_ref[5] = o3d

    @pl.when(j == NK2 - 1)
    def _():
        dc = jnp.sqrt(jnp.float32(D))
        s_star = sstar_ref[...]
        den = jnp.exp(jnp.full((1, 1), o2d) / dc) + \
            jnp.exp(jnp.full((1, 1), o3d) / dc)
        out_ref[...] = (1.0 - jnp.exp(s_star / dc) / den) * s_star


def kernel(queries, keys):
    min_d, s_idx, bstep, s_star = pl.pallas_call(
        _stage1_body,
        grid=(NK,),
        in_specs=[pl.BlockSpec((Q, D), lambda j: (0, 0)),
                  pl.BlockSpec((BK, D), lambda j: (j, 0))],
        out_specs=[pl.BlockSpec((1, Q), lambda j: (0, 0)),
                   pl.BlockSpec((1, 1), lambda j: (0, 0)),
                   pl.BlockSpec((1, 1), lambda j: (0, 0)),
                   pl.BlockSpec((1, 1), lambda j: (0, 0))],
        out_shape=[jax.ShapeDtypeStruct((1, Q), jnp.float32),
                   jax.ShapeDtypeStruct((1, 1), jnp.int32),
                   jax.ShapeDtypeStruct((1, 1), jnp.int32),
                   jax.ShapeDtypeStruct((1, 1), jnp.float32)],
        scratch_shapes=[pltpu.VMEM((1, Q), jnp.float32),
                        pltpu.VMEM((1, Q), jnp.int32)],
        compiler_params=pltpu.CompilerParams(
            dimension_semantics=("arbitrary",)),
    )(queries, keys)

    m_test = jax.lax.dynamic_slice(queries, (s_idx[0, 0], 0), (1, D))
    star_idx = pl.pallas_call(
        _star_body,
        grid_spec=pltpu.PrefetchScalarGridSpec(
            num_scalar_prefetch=1,
            grid=(1,),
            in_specs=[pl.BlockSpec((1, D), lambda i, b: (0, 0)),
                      pl.BlockSpec((BK, D), lambda i, b: (b[0], 0))],
            out_specs=pl.BlockSpec((1, 1), lambda i, b: (0, 0)),
        ),
        out_shape=jax.ShapeDtypeStruct((1, 1), jnp.int32),
    )(bstep.reshape((1,)), m_test, keys)
    m_star = jax.lax.dynamic_slice(keys, (star_idx[0, 0], 0), (1, D))
    score = pl.pallas_call(
        _stage2_body,
        grid=(NK2,),
        in_specs=[pl.BlockSpec((1, D), lambda j: (0, 0)),
                  pl.BlockSpec((1, D), lambda j: (0, 0)),
                  pl.BlockSpec((BK2, D), lambda j: (j, 0)),
                  pl.BlockSpec((1, 1), lambda j: (0, 0))],
        out_specs=pl.BlockSpec((1, 1), lambda j: (0, 0)),
        out_shape=jax.ShapeDtypeStruct((1, 1), jnp.float32),
        scratch_shapes=[pltpu.SMEM((8,), jnp.float32)],
        compiler_params=pltpu.CompilerParams(
            dimension_semantics=("arbitrary",)),
    )(m_test, m_star, keys, s_star)

    return score[0, 0], min_d.reshape(32, 32)


# speedup vs baseline: 5.2344x; 1.0044x over previous
import jax
import jax.numpy as jnp
from jax.experimental import pallas as pl
from jax.experimental.pallas import tpu as pltpu

Q, K, D = 1024, 16384, 1536
BQ, BK = 256, 2048
NQ, NK = Q // BQ, K // BK
BK2 = 2048
NK2 = K // BK2
_INF = float("inf")
_EPS = 1e-12


def _stage1_body(q_ref, k_ref, mind_ref, sidx_ref, bstep_ref, sstar_ref,
                 fmin_ref, fstep_ref):
    j = pl.program_id(0)
    q = q_ref[...]
    NH = 2
    H = BK // NH
    ks = [k_ref[pl.ds(h * H, H), :] for h in range(NH)]
    kqs = [jax.lax.dot_general(kh, q, (((1,), (1,)), ((), ())),
                               preferred_element_type=jnp.float32)
           for kh in ks]
    k2s = [jnp.sum(kh * kh, axis=1, keepdims=True) for kh in ks]
    bmins = [jnp.min(k2 - 2.0 * kq, axis=0, keepdims=True)
             for k2, kq in zip(k2s, kqs)]
    bmin = jnp.minimum(bmins[0], bmins[1])

    @pl.when(j == 0)
    def _():
        fmin_ref[...] = jnp.full((1, Q), _INF, jnp.float32)
        fstep_ref[...] = jnp.zeros((1, Q), jnp.int32)

    old_min = fmin_ref[...]
    old_step = fstep_ref[...]
    take = bmin < old_min
    new_min = jnp.where(take, bmin, old_min)
    new_step = jnp.where(take, j, old_step)
    fmin_ref[...] = new_min
    fstep_ref[...] = new_step

    @pl.when(j == NK - 1)
    def _():
        q2 = jax.lax.dot_general(jnp.ones((1, D), jnp.float32), q * q,
                                 (((1,), (1,)), ((), ())),
                                 precision=jax.lax.Precision.HIGHEST,
                                 preferred_element_type=jnp.float32)
        d2min = q2 + new_min
        mind_ref[...] = jnp.sqrt(jnp.maximum(d2min, _EPS))
        s_val = jnp.max(d2min)
        qio = jax.lax.broadcasted_iota(jnp.int32, (1, Q), 1)
        s_idx = jnp.min(jnp.where(d2min == s_val, qio, Q))
        bstep = jnp.sum(jnp.where(qio == s_idx, new_step, 0))
        sidx_ref[...] = jnp.full((1, 1), s_idx, jnp.int32)
        bstep_ref[...] = jnp.full((1, 1), bstep, jnp.int32)
        sstar_ref[...] = jnp.full((1, 1), jnp.sqrt(jnp.maximum(s_val, _EPS)),
                                  jnp.float32)


def _star_body(bstep_pref, mt_ref, k_ref, star_ref):
    k = k_ref[...]
    mt = mt_ref[...]
    kq = jax.lax.dot_general(k, mt, (((1,), (1,)), ((), ())),
                             preferred_element_type=jnp.float32)
    k2 = jnp.sum(k * k, axis=1, keepdims=True)
    e = k2 - 2.0 * kq
    v = jnp.min(e)
    io = jax.lax.broadcasted_iota(jnp.int32, (BK, 1), 0)
    loc = jnp.min(jnp.where(e == v, io, BK))
    star_ref[...] = jnp.full((1, 1), loc + bstep_pref[0] * BK, jnp.int32)


def _stage2_body(mt_ref, ms_ref, k_ref, sstar_ref, out_ref, cand_ref):
    j = pl.program_id(0)
    k = k_ref[...]
    ms = ms_ref[...]
    mt = mt_ref[...]
    dks = k - ms
    dkt = k - mt
    ds = jnp.sqrt(jnp.maximum(
        jnp.sum(dks * dks, axis=1, keepdims=True), _EPS))
    dt = jnp.sqrt(jnp.maximum(
        jnp.sum(dkt * dkt, axis=1, keepdims=True), _EPS))
    io = jax.lax.broadcasted_iota(jnp.int32, (BK2, 1), 0)

    def top1(dvec):
        v = jnp.min(dvec)
        i1 = jnp.min(jnp.where(dvec == v, io, BK2))
        pay = jnp.sum(jnp.where(io == i1, dt, 0.0))
        return v, pay, i1

    bv1, bd1, i1 = top1(ds)
    ds_b = jnp.where(io == i1, _INF, ds)
    bv2, bd2, i2 = top1(ds_b)
    ds_c = jnp.where(io == i2, _INF, ds_b)
    bv3, bd3, _ = top1(ds_c)

    @pl.when(j == 0)
    def _():
        cand_ref[0] = _INF
        cand_ref[1] = _INF
        cand_ref[2] = _INF
        cand_ref[3] = 0.0
        cand_ref[4] = 0.0
        cand_ref[5] = 0.0

    rv1, rv2, rv3 = cand_ref[0], cand_ref[1], cand_ref[2]
    rd1, rd2, rd3 = cand_ref[3], cand_ref[4], cand_ref[5]

    c1 = bv1 < rv1
    o1v = jnp.where(c1, bv1, rv1)
    o1d = jnp.where(c1, bd1, rd1)
    a2 = bv1 < rv2
    A2v = jnp.where(a2, bv1, rv2)
    A2d = jnp.where(a2, bd1, rd2)
    A3v = jnp.where(a2, jnp.where(bv2 < rv2, bv2, rv2),
                    jnp.where(bv1 < rv3, bv1, rv3))
    A3d = jnp.where(a2, jnp.where(bv2 < rv2, bd2, rd2),
                    jnp.where(bv1 < rv3, bd1, rd3))
    b2c = bv2 < rv1
    B2v = jnp.where(b2c, bv2, rv1)
    B2d = jnp.where(b2c, bd2, rd1)
    B3v = jnp.where(b2c, jnp.where(bv3 < rv1, bv3, rv1),
                    jnp.where(bv2 < rv2, bv2, rv2))
    B3d = jnp.where(b2c, jnp.where(bv3 < rv1, bd3, rd1),
                    jnp.where(bv2 < rv2, bd2, rd2))
    o2v = jnp.where(c1, B2v, A2v)
    o2d = jnp.where(c1, B2d, A2d)
    o3v = jnp.where(c1, B3v, A3v)
    o3d = jnp.where(c1, B3d, A3d)
    cand_ref[0] = o1v
    cand_ref[1] = o2v
    cand_ref[2] = o3v
    cand_ref[3] = o1d
    cand_ref[4] = o2d
    cand_ref[5] = o3d

    @pl.when(j == NK2 - 1)
    def _():
        dc = jnp.sqrt(jnp.float32(D))
        s_star = sstar_ref[...]
        den = jnp.exp(jnp.full((1, 1), o2d) / dc) + \
            jnp.exp(jnp.full((1, 1), o3d) / dc)
        out_ref[...] = (1.0 - jnp.exp(s_star / dc) / den) * s_star


def kernel(queries, keys):
    min_d, s_idx, bstep, s_star = pl.pallas_call(
        _stage1_body,
        grid=(NK,),
        in_specs=[pl.BlockSpec((Q, D), lambda j: (0, 0)),
                  pl.BlockSpec((BK, D), lambda j: (j, 0))],
        out_specs=[pl.BlockSpec((1, Q), lambda j: (0, 0)),
                   pl.BlockSpec((1, 1), lambda j: (0, 0)),
                   pl.BlockSpec((1, 1), lambda j: (0, 0)),
                   pl.BlockSpec((1, 1), lambda j: (0, 0))],
        out_shape=[jax.ShapeDtypeStruct((1, Q), jnp.float32),
                   jax.ShapeDtypeStruct((1, 1), jnp.int32),
                   jax.ShapeDtypeStruct((1, 1), jnp.int32),
                   jax.ShapeDtypeStruct((1, 1), jnp.float32)],
        scratch_shapes=[pltpu.VMEM((1, Q), jnp.float32),
                        pltpu.VMEM((1, Q), jnp.int32)],
        compiler_params=pltpu.CompilerParams(
            dimension_semantics=("arbitrary",)),
    )(queries, keys)

    m_test = jax.lax.dynamic_slice(queries, (s_idx[0, 0], 0), (1, D))
    star_idx = pl.pallas_call(
        _star_body,
        grid_spec=pltpu.PrefetchScalarGridSpec(
            num_scalar_prefetch=1,
            grid=(1,),
            in_specs=[pl.BlockSpec((1, D), lambda i, b: (0, 0)),
                      pl.BlockSpec((BK, D), lambda i, b: (b[0], 0))],
            out_specs=pl.BlockSpec((1, 1), lambda i, b: (0, 0)),
        ),
        out_shape=jax.ShapeDtypeStruct((1, 1), jnp.int32),
    )(bstep.reshape((1,)), m_test, keys)
    m_star = jax.lax.dynamic_slice(keys, (star_idx[0, 0], 0), (1, D))
    score = pl.pallas_call(
        _stage2_body,
        grid=(NK2,),
        in_specs=[pl.BlockSpec((1, D), lambda j: (0, 0)),
                  pl.BlockSpec((1, D), lambda j: (0, 0)),
                  pl.BlockSpec((BK2, D), lambda j: (j, 0)),
                  pl.BlockSpec((1, 1), lambda j: (0, 0))],
        out_specs=pl.BlockSpec((1, 1), lambda j: (0, 0)),
        out_shape=jax.ShapeDtypeStruct((1, 1), jnp.float32),
        scratch_shapes=[pltpu.SMEM((8,), jnp.float32)],
        compiler_params=pltpu.CompilerParams(
            dimension_semantics=("arbitrary",)),
    )(m_test, m_star, keys, s_star)

    return score[0, 0], min_d.reshape(32, 32)


# speedup vs baseline: 5.3168x; 1.0157x over previous
import jax
import jax.numpy as jnp
from jax.experimental import pallas as pl
from jax.experimental.pallas import tpu as pltpu

Q, K, D = 1024, 16384, 1536
BQ, BK = 256, 2048
NQ, NK = Q // BQ, K // BK
BK2 = 2048
NK2 = K // BK2
_INF = float("inf")
_EPS = 1e-12


def _stage1_body(q_ref, k_ref, mind_ref, sidx_ref, bstep_ref, sstar_ref,
                 fmin_ref, fstep_ref):
    j = pl.program_id(0)
    q = q_ref[...]
    NH = 4
    H = BK // NH
    ks = [k_ref[pl.ds(h * H, H), :] for h in range(NH)]
    kqs = [jax.lax.dot_general(kh, q, (((1,), (1,)), ((), ())),
                               preferred_element_type=jnp.float32)
           for kh in ks]
    k2s = [jnp.sum(kh * kh, axis=1, keepdims=True) for kh in ks]
    bmins = [jnp.min(k2 - 2.0 * kq, axis=0, keepdims=True)
             for k2, kq in zip(k2s, kqs)]
    bmin = bmins[0]
    for b in bmins[1:]:
        bmin = jnp.minimum(bmin, b)

    @pl.when(j == 0)
    def _():
        fmin_ref[...] = jnp.full((1, Q), _INF, jnp.float32)
        fstep_ref[...] = jnp.zeros((1, Q), jnp.int32)

    old_min = fmin_ref[...]
    old_step = fstep_ref[...]
    take = bmin < old_min
    new_min = jnp.where(take, bmin, old_min)
    new_step = jnp.where(take, j, old_step)
    fmin_ref[...] = new_min
    fstep_ref[...] = new_step

    @pl.when(j == NK - 1)
    def _():
        q2 = jax.lax.dot_general(jnp.ones((1, D), jnp.float32), q * q,
                                 (((1,), (1,)), ((), ())),
                                 precision=jax.lax.Precision.HIGHEST,
                                 preferred_element_type=jnp.float32)
        d2min = q2 + new_min
        mind_ref[...] = jnp.sqrt(jnp.maximum(d2min, _EPS))
        s_val = jnp.max(d2min)
        qio = jax.lax.broadcasted_iota(jnp.int32, (1, Q), 1)
        s_idx = jnp.min(jnp.where(d2min == s_val, qio, Q))
        bstep = jnp.sum(jnp.where(qio == s_idx, new_step, 0))
        sidx_ref[...] = jnp.full((1, 1), s_idx, jnp.int32)
        bstep_ref[...] = jnp.full((1, 1), bstep, jnp.int32)
        sstar_ref[...] = jnp.full((1, 1), jnp.sqrt(jnp.maximum(s_val, _EPS)),
                                  jnp.float32)


def _star_body(bstep_pref, mt_ref, k_ref, star_ref):
    k = k_ref[...]
    mt = mt_ref[...]
    kq = jax.lax.dot_general(k, mt, (((1,), (1,)), ((), ())),
                             preferred_element_type=jnp.float32)
    k2 = jnp.sum(k * k, axis=1, keepdims=True)
    e = k2 - 2.0 * kq
    v = jnp.min(e)
    io = jax.lax.broadcasted_iota(jnp.int32, (BK, 1), 0)
    loc = jnp.min(jnp.where(e == v, io, BK))
    star_ref[...] = jnp.full((1, 1), loc + bstep_pref[0] * BK, jnp.int32)


def _stage2_body(mt_ref, ms_ref, k_ref, sstar_ref, out_ref, cand_ref):
    j = pl.program_id(0)
    k = k_ref[...]
    ms = ms_ref[...]
    mt = mt_ref[...]
    dks = k - ms
    dkt = k - mt
    ds = jnp.sqrt(jnp.maximum(
        jnp.sum(dks * dks, axis=1, keepdims=True), _EPS))
    dt = jnp.sqrt(jnp.maximum(
        jnp.sum(dkt * dkt, axis=1, keepdims=True), _EPS))
    io = jax.lax.broadcasted_iota(jnp.int32, (BK2, 1), 0)

    def top1(dvec):
        v = jnp.min(dvec)
        i1 = jnp.min(jnp.where(dvec == v, io, BK2))
        pay = jnp.sum(jnp.where(io == i1, dt, 0.0))
        return v, pay, i1

    bv1, bd1, i1 = top1(ds)
    ds_b = jnp.where(io == i1, _INF, ds)
    bv2, bd2, i2 = top1(ds_b)
    ds_c = jnp.where(io == i2, _INF, ds_b)
    bv3, bd3, _ = top1(ds_c)

    @pl.when(j == 0)
    def _():
        cand_ref[0] = _INF
        cand_ref[1] = _INF
        cand_ref[2] = _INF
        cand_ref[3] = 0.0
        cand_ref[4] = 0.0
        cand_ref[5] = 0.0

    rv1, rv2, rv3 = cand_ref[0], cand_ref[1], cand_ref[2]
    rd1, rd2, rd3 = cand_ref[3], cand_ref[4], cand_ref[5]

    c1 = bv1 < rv1
    o1v = jnp.where(c1, bv1, rv1)
    o1d = jnp.where(c1, bd1, rd1)
    a2 = bv1 < rv2
    A2v = jnp.where(a2, bv1, rv2)
    A2d = jnp.where(a2, bd1, rd2)
    A3v = jnp.where(a2, jnp.where(bv2 < rv2, bv2, rv2),
                    jnp.where(bv1 < rv3, bv1, rv3))
    A3d = jnp.where(a2, jnp.where(bv2 < rv2, bd2, rd2),
                    jnp.where(bv1 < rv3, bd1, rd3))
    b2c = bv2 < rv1
    B2v = jnp.where(b2c, bv2, rv1)
    B2d = jnp.where(b2c, bd2, rd1)
    B3v = jnp.where(b2c, jnp.where(bv3 < rv1, bv3, rv1),
                    jnp.where(bv2 < rv2, bv2, rv2))
    B3d = jnp.where(b2c, jnp.where(bv3 < rv1, bd3, rd1),
                    jnp.where(bv2 < rv2, bd2, rd2))
    o2v = jnp.where(c1, B2v, A2v)
    o2d = jnp.where(c1, B2d, A2d)
    o3v = jnp.where(c1, B3v, A3v)
    o3d = jnp.where(c1, B3d, A3d)
    cand_ref[0] = o1v
    cand_ref[1] = o2v
    cand_ref[2] = o3v
    cand_ref[3] = o1d
    cand_ref[4] = o2d
    cand_ref[5] = o3d

    @pl.when(j == NK2 - 1)
    def _():
        dc = jnp.sqrt(jnp.float32(D))
        s_star = sstar_ref[...]
        den = jnp.exp(jnp.full((1, 1), o2d) / dc) + \
            jnp.exp(jnp.full((1, 1), o3d) / dc)
        out_ref[...] = (1.0 - jnp.exp(s_star / dc) / den) * s_star


def kernel(queries, keys):
    min_d, s_idx, bstep, s_star = pl.pallas_call(
        _stage1_body,
        grid=(NK,),
        in_specs=[pl.BlockSpec((Q, D), lambda j: (0, 0)),
                  pl.BlockSpec((BK, D), lambda j: (j, 0))],
        out_specs=[pl.BlockSpec((1, Q), lambda j: (0, 0)),
                   pl.BlockSpec((1, 1), lambda j: (0, 0)),
                   pl.BlockSpec((1, 1), lambda j: (0, 0)),
                   pl.BlockSpec((1, 1), lambda j: (0, 0))],
        out_shape=[jax.ShapeDtypeStruct((1, Q), jnp.float32),
                   jax.ShapeDtypeStruct((1, 1), jnp.int32),
                   jax.ShapeDtypeStruct((1, 1), jnp.int32),
                   jax.ShapeDtypeStruct((1, 1), jnp.float32)],
        scratch_shapes=[pltpu.VMEM((1, Q), jnp.float32),
                        pltpu.VMEM((1, Q), jnp.int32)],
        compiler_params=pltpu.CompilerParams(
            dimension_semantics=("arbitrary",)),
    )(queries, keys)

    m_test = jax.lax.dynamic_slice(queries, (s_idx[0, 0], 0), (1, D))
    star_idx = pl.pallas_call(
        _star_body,
        grid_spec=pltpu.PrefetchScalarGridSpec(
            num_scalar_prefetch=1,
            grid=(1,),
            in_specs=[pl.BlockSpec((1, D), lambda i, b: (0, 0)),
                      pl.BlockSpec((BK, D), lambda i, b: (b[0], 0))],
            out_specs=pl.BlockSpec((1, 1), lambda i, b: (0, 0)),
        ),
        out_shape=jax.ShapeDtypeStruct((1, 1), jnp.int32),
    )(bstep.reshape((1,)), m_test, keys)
    m_star = jax.lax.dynamic_slice(keys, (star_idx[0, 0], 0), (1, D))
    score = pl.pallas_call(
        _stage2_body,
        grid=(NK2,),
        in_specs=[pl.BlockSpec((1, D), lambda j: (0, 0)),
                  pl.BlockSpec((1, D), lambda j: (0, 0)),
                  pl.BlockSpec((BK2, D), lambda j: (j, 0)),
                  pl.BlockSpec((1, 1), lambda j: (0, 0))],
        out_specs=pl.BlockSpec((1, 1), lambda j: (0, 0)),
        out_shape=jax.ShapeDtypeStruct((1, 1), jnp.float32),
        scratch_shapes=[pltpu.SMEM((8,), jnp.float32)],
        compiler_params=pltpu.CompilerParams(
            dimension_semantics=("arbitrary",)),
    )(m_test, m_star, keys, s_star)

    return score[0, 0], min_d.reshape(32, 32)
